# Initial kernel scaffold; baseline (speedup 1.0000x reference)
import jax
import jax.numpy as jnp
from jax.experimental import pallas as pl


def kernel(x, edge_index, W1, a_src1, a_dst1, b1, W2, a_src2, a_dst2, b2, Wc, bc):
    raise NotImplementedError("write your pallas kernel here")


# speedup vs baseline: 1.0913x; 1.0913x over previous
import functools

import jax
import jax.numpy as jnp
from jax.experimental import pallas as pl

N = 10000
E = 320000
IN = 128
HID = 64
HEADS = 8
OUT = 40


def _head_body(emb_ref, wc_ref, bc_ref, logits_ref, soft_ref, hard_ref):
    emb = emb_ref[...]
    logits = emb @ wc_ref[...] + bc_ref[...][None, :]
    mx = jnp.max(logits, axis=1, keepdims=True)
    ex = jnp.exp(logits - mx)
    soft = ex / jnp.sum(ex, axis=1, keepdims=True)
    logits_ref[...] = logits
    soft_ref[...] = soft
    hard_ref[...] = jnp.argmax(logits, axis=1).astype(jnp.int32)[:, None]


def _dense_head(emb, Wc, bc):
    n = emb.shape[0]
    blk = 1000
    grid = (n // blk,)
    return pl.pallas_call(
        _head_body,
        grid=grid,
        in_specs=[
            pl.BlockSpec((blk, HID), lambda i: (i, 0)),
            pl.BlockSpec((HID, OUT), lambda i: (0, 0)),
            pl.BlockSpec((OUT,), lambda i: (0,)),
        ],
        out_specs=[
            pl.BlockSpec((blk, OUT), lambda i: (i, 0)),
            pl.BlockSpec((blk, OUT), lambda i: (i, 0)),
            pl.BlockSpec((blk, 1), lambda i: (i, 0)),
        ],
        out_shape=[
            jax.ShapeDtypeStruct((n, OUT), jnp.float32),
            jax.ShapeDtypeStruct((n, OUT), jnp.float32),
            jax.ShapeDtypeStruct((n, 1), jnp.int32),
        ],
    )(emb, Wc, bc)


def _gat_conv(x, src, dst, W, a_src, a_dst, b, heads, out_ch, concat):
    n = x.shape[0]
    h = (x @ W).reshape(n, heads, out_ch)
    alpha_src = jnp.sum(h * a_src[None], axis=-1)
    alpha_dst = jnp.sum(h * a_dst[None], axis=-1)
    e = alpha_src[src] + alpha_dst[dst]
    e = jax.nn.leaky_relu(e, negative_slope=0.2)
    ex = jnp.exp(e)
    denom = jax.ops.segment_sum(ex, dst, num_segments=n)
    alpha = ex / denom[dst]
    msg = h[src] * alpha[..., None]
    out = jax.ops.segment_sum(msg, dst, num_segments=n)
    if concat:
        out = out.reshape(n, heads * out_ch)
    else:
        out = out.mean(axis=1)
    return out + b


def kernel(x, edge_index, W1, a_src1, a_dst1, b1, W2, a_src2, a_dst2, b2, Wc, bc):
    n = x.shape[0]
    loop = jnp.arange(n, dtype=edge_index.dtype)
    src = jnp.concatenate([edge_index[0], loop])
    dst = jnp.concatenate([edge_index[1], loop])
    h = _gat_conv(x, src, dst, W1, a_src1, a_dst1, b1, HEADS, HID, True)
    h = jax.nn.elu(h)
    emb = _gat_conv(h, src, dst, W2, a_src2, a_dst2, b2, 1, HID, False)
    logits, soft_label, hard = _dense_head(emb, Wc, bc)
    hard_label = hard[:, 0]
    return (logits, emb, soft_label, hard_label)


# speedup vs baseline: 15.1847x; 13.9149x over previous
import functools

import jax
import jax.numpy as jnp
from jax import lax
from jax.experimental import pallas as pl
from jax.experimental.pallas import tpu as pltpu
from jax.experimental.pallas import tpu_sc as plsc

N = 10000
E = 320000
IN = 128
HID = 64
HEADS = 8
OUT = 40

NP = 10240
E2 = E + N
EP = 331776
CH = 128
NW = 32
W_ATT = EP // NW
K_ATT = W_ATT // CH
W_MSG = EP // 16
K_MSG = W_MSG // CH
RPT = NP // 16
NEG = -1e30

_mesh = plsc.VectorSubcoreMesh(core_axis_name="c", subcore_axis_name="s")


def _tc1_body(x_ref, w_ref, am_ref, bm_ref, h0_ref, h1_ref, h2_ref, h3_ref,
              a_ref, b_ref):
    h = jnp.dot(x_ref[...], w_ref[...], preferred_element_type=jnp.float32)
    hp = jax.lax.Precision.HIGHEST
    a_ref[...] = jnp.dot(h, am_ref[...], precision=hp,
                         preferred_element_type=jnp.float32)
    b_ref[...] = jnp.dot(h, bm_ref[...], precision=hp,
                         preferred_element_type=jnp.float32)
    h0_ref[...] = h[:, 0:128]
    h1_ref[...] = h[:, 128:256]
    h2_ref[...] = h[:, 256:384]
    h3_ref[...] = h[:, 384:512]


def _tc1(x, W1, Am, Bm):
    blk = 1000
    f = jnp.float32
    return pl.pallas_call(
        _tc1_body,
        grid=(N // blk,),
        in_specs=[
            pl.BlockSpec((blk, IN), lambda i: (i, 0)),
            pl.BlockSpec((IN, HEADS * HID), lambda i: (0, 0)),
            pl.BlockSpec((HEADS * HID, 16), lambda i: (0, 0)),
            pl.BlockSpec((HEADS * HID, 16), lambda i: (0, 0)),
        ],
        out_specs=[pl.BlockSpec((blk, 128), lambda i: (i, 0))] * 4
        + [pl.BlockSpec((blk, 16), lambda i: (i, 0))] * 2,
        out_shape=[jax.ShapeDtypeStruct((N, 128), f)] * 4
        + [jax.ShapeDtypeStruct((N, 16), f)] * 2,
    )(x, W1, Am, Bm)


def _tc2_body(p0_ref, p1_ref, p2_ref, p3_ref, b1_ref, w2_ref, am_ref, bm_ref,
              h_ref, a_ref, b_ref):
    x2 = jnp.concatenate(
        [p0_ref[...], p1_ref[...], p2_ref[...], p3_ref[...]], axis=1)
    x2 = x2 + b1_ref[...][None, :]
    x2 = jnp.where(x2 > 0, x2, jnp.exp(x2) - 1.0)
    h = jnp.dot(x2, w2_ref[...], preferred_element_type=jnp.float32)
    h_ref[...] = h
    hp = jax.lax.Precision.HIGHEST
    a_ref[...] = jnp.dot(h, am_ref[...], precision=hp,
                         preferred_element_type=jnp.float32)
    b_ref[...] = jnp.dot(h, bm_ref[...], precision=hp,
                         preferred_element_type=jnp.float32)


def _tc2(p0, p1, p2, p3, b1, W2, Am, Bm):
    blk = 1000
    f = jnp.float32
    return pl.pallas_call(
        _tc2_body,
        grid=(N // blk,),
        in_specs=[pl.BlockSpec((blk, 128), lambda i: (i, 0))] * 4
        + [
            pl.BlockSpec((HEADS * HID,), lambda i: (0,)),
            pl.BlockSpec((HEADS * HID, HID), lambda i: (0, 0)),
            pl.BlockSpec((HID, 16), lambda i: (0, 0)),
            pl.BlockSpec((HID, 16), lambda i: (0, 0)),
        ],
        out_specs=[
            pl.BlockSpec((blk, HID), lambda i: (i, 0)),
            pl.BlockSpec((blk, 16), lambda i: (i, 0)),
            pl.BlockSpec((blk, 16), lambda i: (i, 0)),
        ],
        out_shape=[
            jax.ShapeDtypeStruct((N, HID), f),
            jax.ShapeDtypeStruct((N, 16), f),
            jax.ShapeDtypeStruct((N, 16), f),
        ],
    )(p0, p1, p2, p3, b1, W2, Am, Bm)


def _tc3_body(oa_ref, ob_ref, b2_ref, wc_ref, bc_ref,
              emb_ref, logits_ref, soft_ref, hard_ref):
    emb = oa_ref[...] + ob_ref[...] + b2_ref[...][None, :]
    logits = jnp.dot(emb, wc_ref[...], preferred_element_type=jnp.float32)
    logits = logits + bc_ref[...][None, :]
    mx = jnp.max(logits, axis=1, keepdims=True)
    ex = jnp.exp(logits - mx)
    soft = ex / jnp.sum(ex, axis=1, keepdims=True)
    emb_ref[...] = emb
    logits_ref[...] = logits
    soft_ref[...] = soft
    hard_ref[...] = jnp.argmax(soft, axis=1).astype(jnp.int32)[:, None]


def _tc3(oa, ob, b2, Wc, bc):
    blk = 1000
    f = jnp.float32
    return pl.pallas_call(
        _tc3_body,
        grid=(N // blk,),
        in_specs=[
            pl.BlockSpec((blk, HID), lambda i: (i, 0)),
            pl.BlockSpec((blk, HID), lambda i: (i, 0)),
            pl.BlockSpec((HID,), lambda i: (0,)),
            pl.BlockSpec((HID, OUT), lambda i: (0, 0)),
            pl.BlockSpec((OUT,), lambda i: (0,)),
        ],
        out_specs=[
            pl.BlockSpec((blk, HID), lambda i: (i, 0)),
            pl.BlockSpec((blk, OUT), lambda i: (i, 0)),
            pl.BlockSpec((blk, OUT), lambda i: (i, 0)),
            pl.BlockSpec((blk, 1), lambda i: (i, 0)),
        ],
        out_shape=[
            jax.ShapeDtypeStruct((N, HID), f),
            jax.ShapeDtypeStruct((N, OUT), f),
            jax.ShapeDtypeStruct((N, OUT), f),
            jax.ShapeDtypeStruct((N, 1), jnp.int32),
        ],
    )(oa, ob, b2, Wc, bc)


def _att_body(srcp, dstp, A, B, ex_out, den_out,
              idxs, idxd, av, bv, zb, den_acc, sem):
    c = lax.axis_index("c")
    s = lax.axis_index("s")
    w = c * 16 + s

    @pl.loop(0, 16)
    def _zb(i):
        zb[i, :] = jnp.zeros((16,), jnp.float32)

    @pl.loop(0, RPT // 16)
    def _zero(i):
        pltpu.sync_copy(zb, den_acc.at[pl.ds(s * RPT + i * 16, 16)])

    plsc.subcore_barrier()

    @pl.loop(0, K_ATT)
    def _chunk(k):
        base = w * W_ATT + k * CH
        pltpu.sync_copy(srcp.at[pl.ds(base, CH)], idxs)
        pltpu.sync_copy(dstp.at[pl.ds(base, CH)], idxd)
        pltpu.async_copy(A.at[idxs], av, sem).wait()
        pltpu.async_copy(B.at[idxd], bv, sem).wait()

        @pl.loop(0, CH)
        def _edge(j):
            e = av[j, :] + bv[j, :]
            e = jnp.maximum(e, 0.2 * e)
            av[j, :] = jnp.exp(e)

        pltpu.sync_copy(av, ex_out.at[pl.ds(base, CH)])
        pltpu.sync_copy(av, den_acc.at[idxd], add=True)

    plsc.subcore_barrier()
    pltpu.sync_copy(den_acc.at[pl.ds(s * RPT, RPT)],
                    den_out.at[c, pl.ds(s * RPT, RPT)])


def _att(srcp, dstp, A, B):
    f = jnp.float32
    return pl.kernel(
        _att_body,
        out_type=[
            jax.ShapeDtypeStruct((EP, 16), f),
            jax.ShapeDtypeStruct((2, NP, 16), f),
        ],
        mesh=_mesh,
        compiler_params=pltpu.CompilerParams(use_tc_tiling_on_sc=False, needs_layout_passes=False),
        scratch_types=[
            pltpu.VMEM((CH,), jnp.int32),
            pltpu.VMEM((CH,), jnp.int32),
            pltpu.VMEM((CH, 16), f),
            pltpu.VMEM((CH, 16), f),
            pltpu.VMEM((16, 16), f),
            pltpu.VMEM_SHARED((NP, 16), f),
            pltpu.SemaphoreType.DMA,
        ],
    )(srcp, dstp, A, B)


def _alpha_body(dstp, exr, d0, d1, al_out, idxd, exv, dv0, dv1, sem):
    c = lax.axis_index("c")
    s = lax.axis_index("s")
    w = c * 16 + s

    @pl.loop(0, K_ATT)
    def _chunk(k):
        base = w * W_ATT + k * CH
        pltpu.sync_copy(dstp.at[pl.ds(base, CH)], idxd)
        pltpu.sync_copy(exr.at[pl.ds(base, CH)], exv)
        pltpu.async_copy(d0.at[idxd], dv0, sem).wait()
        pltpu.async_copy(d1.at[idxd], dv1, sem).wait()

        @pl.loop(0, CH)
        def _edge(j):
            den = dv0[j, :] + dv1[j, :] + 1e-16
            exv[j, :] = exv[j, :] / den

        pltpu.sync_copy(exv, al_out.at[pl.ds(base, CH)])


def _alpha(dstp, exr, d0, d1):
    f = jnp.float32
    return pl.kernel(
        _alpha_body,
        out_type=jax.ShapeDtypeStruct((EP, 16), f),
        mesh=_mesh,
        compiler_params=pltpu.CompilerParams(use_tc_tiling_on_sc=False, needs_layout_passes=False),
        scratch_types=[
            pltpu.VMEM((CH,), jnp.int32),
            pltpu.VMEM((CH, 16), f),
            pltpu.VMEM((CH, 16), f),
            pltpu.VMEM((CH, 16), f),
            pltpu.SemaphoreType.DMA,
        ],
    )(dstp, exr, d0, d1)


def _msg1_body(srcp, dstp, hflat, alr, out_hbm,
               idxs, idxd, idx2, hv, alv, zb, acc, sem):
    c = lax.axis_index("c")
    s = lax.axis_index("s")

    @pl.loop(0, 16)
    def _zb(i):
        @pl.loop(0, 8)
        def _zbf(fb):
            zb[i, pl.ds(fb * 16, 16)] = jnp.zeros((16,), jnp.float32)

    for q in (0, 1):
        P = 2 * c + q
        off = P * NP

        @pl.loop(0, RPT // 16)
        def _zero(i):
            pltpu.sync_copy(zb, acc.at[pl.ds(s * RPT + i * 16, 16)])

        plsc.subcore_barrier()

        @pl.loop(0, K_MSG)
        def _chunk(k):
            base = s * W_MSG + k * CH
            pltpu.sync_copy(srcp.at[pl.ds(base, CH)], idxs)
            pltpu.sync_copy(dstp.at[pl.ds(base, CH)], idxd)

            @pl.loop(0, CH // 16)
            def _ix(j):
                idx2[pl.ds(j * 16, 16)] = idxs[pl.ds(j * 16, 16)] + off

            pltpu.async_copy(hflat.at[idx2], hv, sem).wait()
            pltpu.sync_copy(alr.at[pl.ds(base, CH)], alv)
            l0 = 2 * P

            @pl.loop(0, CH)
            def _edge(j):
                ji = jnp.full((16,), j, jnp.int32)
                a0 = plsc.load_gather(alv, [ji, jnp.full((16,), l0, jnp.int32)])
                a1 = plsc.load_gather(alv, [ji, jnp.full((16,), l0 + 1,
                                                         jnp.int32)])
                for fb in range(4):
                    sl = pl.ds(fb * 16, 16)
                    hv[j, sl] = hv[j, sl] * a0
                for fb in range(4, 8):
                    sl = pl.ds(fb * 16, 16)
                    hv[j, sl] = hv[j, sl] * a1

            pltpu.sync_copy(hv, acc.at[idxd], add=True)

        plsc.subcore_barrier()
        pltpu.sync_copy(acc.at[pl.ds(s * RPT, RPT)],
                        out_hbm.at[P, pl.ds(s * RPT, RPT)])


def _msg1(srcp, dstp, hflat, alr):
    f = jnp.float32
    return pl.kernel(
        _msg1_body,
        out_type=jax.ShapeDtypeStruct((4, NP, 128), f),
        mesh=_mesh,
        compiler_params=pltpu.CompilerParams(use_tc_tiling_on_sc=False, needs_layout_passes=False),
        scratch_types=[
            pltpu.VMEM((CH,), jnp.int32),
            pltpu.VMEM((CH,), jnp.int32),
            pltpu.VMEM((CH,), jnp.int32),
            pltpu.VMEM((CH, 128), f),
            pltpu.VMEM((CH, 16), f),
            pltpu.VMEM((16, 128), f),
            pltpu.VMEM_SHARED((NP, 128), f),
            pltpu.SemaphoreType.DMA,
        ],
    )(srcp, dstp, hflat, alr)


def _msg2_body(srcp, dstp, h2, alr, out_hbm,
               idxs, idxd, hv, alv, zb, acc, sem):
    c = lax.axis_index("c")
    s = lax.axis_index("s")
    w = c * 16 + s

    @pl.loop(0, 16)
    def _zb(i):
        @pl.loop(0, 4)
        def _zbf(fb):
            zb[i, pl.ds(fb * 16, 16)] = jnp.zeros((16,), jnp.float32)

    @pl.loop(0, RPT // 16)
    def _zero(i):
        pltpu.sync_copy(zb, acc.at[pl.ds(s * RPT + i * 16, 16)])

    plsc.subcore_barrier()

    @pl.loop(0, K_ATT)
    def _chunk(k):
        base = w * W_ATT + k * CH
        pltpu.sync_copy(srcp.at[pl.ds(base, CH)], idxs)
        pltpu.sync_copy(dstp.at[pl.ds(base, CH)], idxd)
        pltpu.async_copy(h2.at[idxs], hv, sem).wait()
        pltpu.sync_copy(alr.at[pl.ds(base, CH)], alv)

        @pl.loop(0, CH)
        def _edge(j):
            ji = jnp.full((16,), j, jnp.int32)
            a0 = plsc.load_gather(alv, [ji, jnp.zeros((16,), jnp.int32)])
            for fb in range(4):
                sl = pl.ds(fb * 16, 16)
                hv[j, sl] = hv[j, sl] * a0

        pltpu.sync_copy(hv, acc.at[idxd], add=True)

    plsc.subcore_barrier()
    pltpu.sync_copy(acc.at[pl.ds(s * RPT, RPT)],
                    out_hbm.at[c, pl.ds(s * RPT, RPT)])


def _msg2(srcp, dstp, h2, alr):
    f = jnp.float32
    return pl.kernel(
        _msg2_body,
        out_type=jax.ShapeDtypeStruct((2, NP, HID), f),
        mesh=_mesh,
        compiler_params=pltpu.CompilerParams(use_tc_tiling_on_sc=False, needs_layout_passes=False),
        scratch_types=[
            pltpu.VMEM((CH,), jnp.int32),
            pltpu.VMEM((CH,), jnp.int32),
            pltpu.VMEM((CH, HID), f),
            pltpu.VMEM((CH, 16), f),
            pltpu.VMEM((16, HID), f),
            pltpu.VMEM_SHARED((NP, HID), f),
            pltpu.SemaphoreType.DMA,
        ],
    )(srcp, dstp, h2, alr)


def _att_mats(a_src, a_dst, heads, hid):
    eye = jnp.eye(heads, 16, dtype=jnp.float32)
    Am = jnp.einsum("hk,hc->hkc", a_src, eye).reshape(heads * hid, 16)
    Bm = jnp.einsum("hk,hc->hkc", a_dst, eye).reshape(heads * hid, 16)
    return Am, Bm


def kernel(x, edge_index, W1, a_src1, a_dst1, b1, W2, a_src2, a_dst2, b2,
           Wc, bc):
    loop = jnp.arange(N, dtype=jnp.int32)
    padi = jnp.full((EP - E2,), N, jnp.int32)
    srcp = jnp.concatenate([edge_index[0].astype(jnp.int32), loop, padi])
    dstp = jnp.concatenate([edge_index[1].astype(jnp.int32), loop, padi])

    Am1, Bm1 = _att_mats(a_src1, a_dst1, HEADS, HID)
    Am2, Bm2 = _att_mats(a_src2, a_dst2, 1, HID)

    p0, p1, p2, p3, A1r, B1r = _tc1(x, W1, Am1, Bm1)
    padA = jnp.full((NP - N, 16), NEG, jnp.float32)
    padH = jnp.zeros((NP - N, 128), jnp.float32)
    A1 = jnp.concatenate([A1r, padA])
    B1 = jnp.concatenate([B1r, padA])
    hflat = jnp.concatenate([p0, padH, p1, padH, p2, padH, p3, padH])

    ex1, den1 = _att(srcp, dstp, A1, B1)
    al1 = _alpha(dstp, ex1, den1[0], den1[1])
    out1 = _msg1(srcp, dstp, hflat, al1)

    h2r, A2r, B2r = _tc2(out1[0, :N], out1[1, :N], out1[2, :N], out1[3, :N],
                         b1, W2, Am2, Bm2)
    h2 = jnp.concatenate([h2r, jnp.zeros((NP - N, HID), jnp.float32)])
    A2 = jnp.concatenate([A2r, padA])
    B2 = jnp.concatenate([B2r, padA])

    ex2, den2 = _att(srcp, dstp, A2, B2)
    al2 = _alpha(dstp, ex2, den2[0], den2[1])
    out2 = _msg2(srcp, dstp, h2, al2)

    emb, logits, soft, hard = _tc3(out2[0, :N], out2[1, :N], b2, Wc, bc)
    return (logits, emb, soft, hard[:, 0])


# speedup vs baseline: 18.8858x; 1.2437x over previous
import functools

import jax
import jax.numpy as jnp
from jax import lax
from jax.experimental import pallas as pl
from jax.experimental.pallas import tpu as pltpu
from jax.experimental.pallas import tpu_sc as plsc

N = 10000
E = 320000
IN = 128
HID = 64
HEADS = 8
OUT = 40

NP = 10240
E2 = E + N
EP = 335872
CH = 128
NW = 32
W_ATT = EP // NW
K_ATT = W_ATT // CH
W_MSG = EP // 16
CM = 64
K_MSG = W_MSG // CM
RPT = NP // 16
NEG = -1e30

_mesh = plsc.VectorSubcoreMesh(core_axis_name="c", subcore_axis_name="s")
_sc_params = pltpu.CompilerParams(use_tc_tiling_on_sc=False,
                                  needs_layout_passes=False)


def _tc1_body(x_ref, w_ref, am_ref, bm_ref, h0_ref, h1_ref, h2_ref, h3_ref,
              a_ref, b_ref):
    h = jnp.dot(x_ref[...], w_ref[...], preferred_element_type=jnp.float32)
    hp = jax.lax.Precision.HIGHEST
    a_ref[...] = jnp.dot(h, am_ref[...], precision=hp,
                         preferred_element_type=jnp.float32)
    b_ref[...] = jnp.dot(h, bm_ref[...], precision=hp,
                         preferred_element_type=jnp.float32)
    h0_ref[...] = h[:, 0:128]
    h1_ref[...] = h[:, 128:256]
    h2_ref[...] = h[:, 256:384]
    h3_ref[...] = h[:, 384:512]


def _tc1(x, W1, Am, Bm):
    blk = 1000
    f = jnp.float32
    return pl.pallas_call(
        _tc1_body,
        grid=(N // blk,),
        in_specs=[
            pl.BlockSpec((blk, IN), lambda i: (i, 0)),
            pl.BlockSpec((IN, HEADS * HID), lambda i: (0, 0)),
            pl.BlockSpec((HEADS * HID, 16), lambda i: (0, 0)),
            pl.BlockSpec((HEADS * HID, 16), lambda i: (0, 0)),
        ],
        out_specs=[pl.BlockSpec((blk, 128), lambda i: (i, 0))] * 4
        + [pl.BlockSpec((blk, 16), lambda i: (i, 0))] * 2,
        out_shape=[jax.ShapeDtypeStruct((N, 128), f)] * 4
        + [jax.ShapeDtypeStruct((N, 16), f)] * 2,
    )(x, W1, Am, Bm)


def _tc2_body(p0_ref, p1_ref, p2_ref, p3_ref, b1_ref, w2_ref, am_ref, bm_ref,
              h_ref, a_ref, b_ref):
    x2 = jnp.concatenate(
        [p0_ref[...], p1_ref[...], p2_ref[...], p3_ref[...]], axis=1)
    x2 = x2 + b1_ref[...][None, :]
    x2 = jnp.where(x2 > 0, x2, jnp.exp(x2) - 1.0)
    h = jnp.dot(x2, w2_ref[...], preferred_element_type=jnp.float32)
    h_ref[...] = h
    hp = jax.lax.Precision.HIGHEST
    a_ref[...] = jnp.dot(h, am_ref[...], precision=hp,
                         preferred_element_type=jnp.float32)
    b_ref[...] = jnp.dot(h, bm_ref[...], precision=hp,
                         preferred_element_type=jnp.float32)


def _tc2(p0, p1, p2, p3, b1, W2, Am, Bm):
    blk = 1000
    f = jnp.float32
    return pl.pallas_call(
        _tc2_body,
        grid=(N // blk,),
        in_specs=[pl.BlockSpec((blk, 128), lambda i: (i, 0))] * 4
        + [
            pl.BlockSpec((HEADS * HID,), lambda i: (0,)),
            pl.BlockSpec((HEADS * HID, HID), lambda i: (0, 0)),
            pl.BlockSpec((HID, 16), lambda i: (0, 0)),
            pl.BlockSpec((HID, 16), lambda i: (0, 0)),
        ],
        out_specs=[
            pl.BlockSpec((blk, HID), lambda i: (i, 0)),
            pl.BlockSpec((blk, 16), lambda i: (i, 0)),
            pl.BlockSpec((blk, 16), lambda i: (i, 0)),
        ],
        out_shape=[
            jax.ShapeDtypeStruct((N, HID), f),
            jax.ShapeDtypeStruct((N, 16), f),
            jax.ShapeDtypeStruct((N, 16), f),
        ],
    )(p0, p1, p2, p3, b1, W2, Am, Bm)


def _tc3_body(oa_ref, ob_ref, b2_ref, wc_ref, bc_ref,
              emb_ref, logits_ref, soft_ref, hard_ref):
    emb = oa_ref[...] + ob_ref[...] + b2_ref[...][None, :]
    logits = jnp.dot(emb, wc_ref[...], preferred_element_type=jnp.float32)
    logits = logits + bc_ref[...][None, :]
    mx = jnp.max(logits, axis=1, keepdims=True)
    ex = jnp.exp(logits - mx)
    soft = ex / jnp.sum(ex, axis=1, keepdims=True)
    emb_ref[...] = emb
    logits_ref[...] = logits
    soft_ref[...] = soft
    hard_ref[...] = jnp.argmax(soft, axis=1).astype(jnp.int32)[:, None]


def _tc3(oa, ob, b2, Wc, bc):
    blk = 1000
    f = jnp.float32
    return pl.pallas_call(
        _tc3_body,
        grid=(N // blk,),
        in_specs=[
            pl.BlockSpec((blk, HID), lambda i: (i, 0)),
            pl.BlockSpec((blk, HID), lambda i: (i, 0)),
            pl.BlockSpec((HID,), lambda i: (0,)),
            pl.BlockSpec((HID, OUT), lambda i: (0, 0)),
            pl.BlockSpec((OUT,), lambda i: (0,)),
        ],
        out_specs=[
            pl.BlockSpec((blk, HID), lambda i: (i, 0)),
            pl.BlockSpec((blk, OUT), lambda i: (i, 0)),
            pl.BlockSpec((blk, OUT), lambda i: (i, 0)),
            pl.BlockSpec((blk, 1), lambda i: (i, 0)),
        ],
        out_shape=[
            jax.ShapeDtypeStruct((N, HID), f),
            jax.ShapeDtypeStruct((N, OUT), f),
            jax.ShapeDtypeStruct((N, OUT), f),
            jax.ShapeDtypeStruct((N, 1), jnp.int32),
        ],
    )(oa, ob, b2, Wc, bc)


def _zero_acc16(zb, acc, s, width_copies, sem):
    del sem

    @pl.loop(0, width_copies)
    def _z(i):
        pltpu.sync_copy(zb, acc.at[pl.ds(s * RPT + i * 16, 16)])


def _att_body(srcp, dstp, A, B, ex_out, den_out,
              idxs0, idxs1, idxd0, idxd1,
              av0, av1, bv0, bv1, ev0, ev1, zb, den_acc,
              sg0, sg1):
    c = lax.axis_index("c")
    s = lax.axis_index("s")
    w = c * 16 + s
    idxs = (idxs0, idxs1)
    idxd = (idxd0, idxd1)
    av = (av0, av1)
    bv = (bv0, bv1)
    ev = (ev0, ev1)
    sg = (sg0, sg1)

    @pl.loop(0, 16)
    def _zb(i):
        zb[i, :] = jnp.zeros((16,), jnp.float32)

    _zero_acc16(zb, den_acc, s, RPT // 16, None)
    plsc.subcore_barrier()

    def load_chunk(kc, b):
        base = w * W_ATT + kc * CH
        pltpu.sync_copy(srcp.at[pl.ds(base, CH)], idxs[b])
        pltpu.sync_copy(dstp.at[pl.ds(base, CH)], idxd[b])
        pltpu.async_copy(A.at[idxs[b]], av[b], sg[b])
        pltpu.async_copy(B.at[idxd[b]], bv[b], sg[b])

    for b in (0, 1):
        load_chunk(b, b)

    @pl.loop(0, K_ATT, step=2)
    def _g(g):
        for b in (0, 1):
            kc = g + b
            pltpu.make_async_copy(A.at[idxs[b]], av[b], sg[b]).wait()
            pltpu.make_async_copy(B.at[idxd[b]], bv[b], sg[b]).wait()

            @pl.loop(0, CH, unroll=4)
            def _edge(j):
                e = av[b][j, :] + bv[b][j, :]
                e = jnp.maximum(e, 0.2 * e)
                ev[b][j, :] = jnp.exp(e)

            base = w * W_ATT + kc * CH
            pltpu.sync_copy(ev[b], ex_out.at[pl.ds(base, CH)])
            pltpu.sync_copy(ev[b], den_acc.at[idxd[b]], add=True)

            @pl.when(kc + 2 < K_ATT)
            def _():
                load_chunk(kc + 2, b)

    plsc.subcore_barrier()
    pltpu.sync_copy(den_acc.at[pl.ds(s * RPT, RPT)],
                    den_out.at[c, pl.ds(s * RPT, RPT)])


def _att(srcp, dstp, A, B):
    f = jnp.float32
    return pl.kernel(
        _att_body,
        out_type=[
            jax.ShapeDtypeStruct((EP, 16), f),
            jax.ShapeDtypeStruct((2, NP, 16), f),
        ],
        mesh=_mesh,
        compiler_params=_sc_params,
        scratch_types=(
            [pltpu.VMEM((CH,), jnp.int32)] * 4
            + [pltpu.VMEM((CH, 16), f)] * 6
            + [pltpu.VMEM((16, 16), f),
               pltpu.VMEM_SHARED((NP, 16), f)]
            + [pltpu.SemaphoreType.DMA] * 2
        ),
    )(srcp, dstp, A, B)


def _alpha_body(dstp, exr, d0, d1, al_out,
                idxd0, idxd1, ev0, ev1, dv00, dv01, dv10, dv11,
                sg0, sg1):
    c = lax.axis_index("c")
    s = lax.axis_index("s")
    w = c * 16 + s
    idxd = (idxd0, idxd1)
    ev = (ev0, ev1)
    dv0 = (dv00, dv01)
    dv1 = (dv10, dv11)
    sg = (sg0, sg1)

    def load_chunk(kc, b):
        base = w * W_ATT + kc * CH
        pltpu.sync_copy(dstp.at[pl.ds(base, CH)], idxd[b])
        pltpu.async_copy(exr.at[pl.ds(base, CH)], ev[b], sg[b])
        pltpu.async_copy(d0.at[idxd[b]], dv0[b], sg[b])
        pltpu.async_copy(d1.at[idxd[b]], dv1[b], sg[b])

    for b in (0, 1):
        load_chunk(b, b)

    @pl.loop(0, K_ATT, step=2)
    def _g(g):
        for b in (0, 1):
            kc = g + b
            pltpu.make_async_copy(exr.at[pl.ds(0, CH)], ev[b], sg[b]).wait()
            pltpu.make_async_copy(d0.at[idxd[b]], dv0[b], sg[b]).wait()
            pltpu.make_async_copy(d1.at[idxd[b]], dv1[b], sg[b]).wait()

            @pl.loop(0, CH, unroll=4)
            def _edge(j):
                den = dv0[b][j, :] + dv1[b][j, :] + 1e-16
                ev[b][j, :] = ev[b][j, :] / den

            base = w * W_ATT + kc * CH
            pltpu.sync_copy(ev[b], al_out.at[pl.ds(base, CH)])

            @pl.when(kc + 2 < K_ATT)
            def _():
                load_chunk(kc + 2, b)


def _alpha(dstp, exr, d0, d1):
    f = jnp.float32
    return pl.kernel(
        _alpha_body,
        out_type=jax.ShapeDtypeStruct((EP, 16), f),
        mesh=_mesh,
        compiler_params=_sc_params,
        scratch_types=(
            [pltpu.VMEM((CH,), jnp.int32)] * 2
            + [pltpu.VMEM((CH, 16), f)] * 6
            + [pltpu.SemaphoreType.DMA] * 2
        ),
    )(dstp, exr, d0, d1)


def _msg1_body(srcp, dstp, hflat, alr, out_hbm,
               idxs0, idxs1, idxd0, idxd1, ix20, ix21,
               hv0, hv1, alv0, alv1, zb, acc,
               sg0, sg1):
    c = lax.axis_index("c")
    s = lax.axis_index("s")
    idxs = (idxs0, idxs1)
    idxd = (idxd0, idxd1)
    ix2 = (ix20, ix21)
    hv = (hv0, hv1)
    alv = (alv0, alv1)
    sg = (sg0, sg1)

    @pl.loop(0, 16)
    def _zb(i):
        @pl.loop(0, 8)
        def _zbf(fb):
            zb[i, pl.ds(fb * 16, 16)] = jnp.zeros((16,), jnp.float32)

    for q in (0, 1):
        P = 2 * c + q
        off = P * NP
        l0 = 2 * P

        _zero_acc16(zb, acc, s, RPT // 16, None)
        plsc.subcore_barrier()

        def load_chunk(kc, b, off=off):
            base = s * W_MSG + kc * CM
            pltpu.sync_copy(srcp.at[pl.ds(base, CM)], idxs[b])
            pltpu.sync_copy(dstp.at[pl.ds(base, CM)], idxd[b])

            @pl.loop(0, CM // 16)
            def _ix(i):
                sl = pl.ds(i * 16, 16)
                ix2[b][sl] = idxs[b][sl] + off

            pltpu.async_copy(hflat.at[ix2[b]], hv[b], sg[b])
            pltpu.async_copy(alr.at[pl.ds(base, CM)], alv[b], sg[b])

        for b in (0, 1):
            load_chunk(b, b)

        @pl.loop(0, K_MSG, step=2)
        def _g(g):
            for b in (0, 1):
                kc = g + b
                pltpu.make_async_copy(hflat.at[ix2[b]], hv[b], sg[b]).wait()
                pltpu.make_async_copy(alr.at[pl.ds(0, CM)], alv[b],
                                      sg[b]).wait()

                @pl.loop(0, CM, unroll=2)
                def _edge(j):
                    ji = jnp.full((16,), j, jnp.int32)
                    a0 = plsc.load_gather(
                        alv[b], [ji, jnp.full((16,), l0, jnp.int32)])
                    a1 = plsc.load_gather(
                        alv[b], [ji, jnp.full((16,), l0 + 1, jnp.int32)])
                    for fb in range(4):
                        sl = pl.ds(fb * 16, 16)
                        hv[b][j, sl] = hv[b][j, sl] * a0
                    for fb in range(4, 8):
                        sl = pl.ds(fb * 16, 16)
                        hv[b][j, sl] = hv[b][j, sl] * a1

                pltpu.sync_copy(hv[b], acc.at[idxd[b]], add=True)

                @pl.when(kc + 2 < K_MSG)
                def _():
                    load_chunk(kc + 2, b)

        plsc.subcore_barrier()
        pltpu.sync_copy(acc.at[pl.ds(s * RPT, RPT)],
                        out_hbm.at[P, pl.ds(s * RPT, RPT)])
        plsc.subcore_barrier()


def _msg1(srcp, dstp, hflat, alr):
    f = jnp.float32
    return pl.kernel(
        _msg1_body,
        out_type=jax.ShapeDtypeStruct((4, NP, 128), f),
        mesh=_mesh,
        compiler_params=_sc_params,
        scratch_types=(
            [pltpu.VMEM((CM,), jnp.int32)] * 6
            + [pltpu.VMEM((CM, 128), f)] * 2
            + [pltpu.VMEM((CM, 16), f)] * 2
            + [pltpu.VMEM((16, 128), f),
               pltpu.VMEM_SHARED((NP, 128), f)]
            + [pltpu.SemaphoreType.DMA] * 2
        ),
    )(srcp, dstp, hflat, alr)


def _msg2_body(srcp, dstp, h2, alr, out_hbm,
               idxs0, idxs1, idxd0, idxd1,
               hv0, hv1, alv0, alv1, zb, acc,
               sg0, sg1):
    c = lax.axis_index("c")
    s = lax.axis_index("s")
    w = c * 16 + s
    idxs = (idxs0, idxs1)
    idxd = (idxd0, idxd1)
    hv = (hv0, hv1)
    alv = (alv0, alv1)
    sg = (sg0, sg1)

    @pl.loop(0, 16)
    def _zb(i):
        @pl.loop(0, 4)
        def _zbf(fb):
            zb[i, pl.ds(fb * 16, 16)] = jnp.zeros((16,), jnp.float32)

    _zero_acc16(zb, acc, s, RPT // 16, None)
    plsc.subcore_barrier()

    def load_chunk(kc, b):
        base = w * W_ATT + kc * CH
        pltpu.sync_copy(srcp.at[pl.ds(base, CH)], idxs[b])
        pltpu.sync_copy(dstp.at[pl.ds(base, CH)], idxd[b])
        pltpu.async_copy(h2.at[idxs[b]], hv[b], sg[b])
        pltpu.async_copy(alr.at[pl.ds(base, CH)], alv[b], sg[b])

    for b in (0, 1):
        load_chunk(b, b)

    @pl.loop(0, K_ATT, step=2)
    def _g(g):
        for b in (0, 1):
            kc = g + b
            pltpu.make_async_copy(h2.at[idxs[b]], hv[b], sg[b]).wait()
            pltpu.make_async_copy(alr.at[pl.ds(0, CH)], alv[b], sg[b]).wait()

            @pl.loop(0, CH, unroll=2)
            def _edge(j):
                ji = jnp.full((16,), j, jnp.int32)
                a0 = plsc.load_gather(
                    alv[b], [ji, jnp.zeros((16,), jnp.int32)])
                for fb in range(4):
                    sl = pl.ds(fb * 16, 16)
                    hv[b][j, sl] = hv[b][j, sl] * a0

            pltpu.sync_copy(hv[b], acc.at[idxd[b]], add=True)

            @pl.when(kc + 2 < K_ATT)
            def _():
                load_chunk(kc + 2, b)

    plsc.subcore_barrier()
    pltpu.sync_copy(acc.at[pl.ds(s * RPT, RPT)],
                    out_hbm.at[c, pl.ds(s * RPT, RPT)])


def _msg2(srcp, dstp, h2, alr):
    f = jnp.float32
    return pl.kernel(
        _msg2_body,
        out_type=jax.ShapeDtypeStruct((2, NP, HID), f),
        mesh=_mesh,
        compiler_params=_sc_params,
        scratch_types=(
            [pltpu.VMEM((CH,), jnp.int32)] * 4
            + [pltpu.VMEM((CH, HID), f)] * 2
            + [pltpu.VMEM((CH, 16), f)] * 2
            + [pltpu.VMEM((16, HID), f),
               pltpu.VMEM_SHARED((NP, HID), f)]
            + [pltpu.SemaphoreType.DMA] * 2
        ),
    )(srcp, dstp, h2, alr)


def _att_mats(a_src, a_dst, heads, hid):
    eye = jnp.eye(heads, 16, dtype=jnp.float32)
    Am = jnp.einsum("hk,hc->hkc", a_src, eye).reshape(heads * hid, 16)
    Bm = jnp.einsum("hk,hc->hkc", a_dst, eye).reshape(heads * hid, 16)
    return Am, Bm


def kernel(x, edge_index, W1, a_src1, a_dst1, b1, W2, a_src2, a_dst2, b2,
           Wc, bc):
    loop = jnp.arange(N, dtype=jnp.int32)
    padi = jnp.full((EP - E2,), N, jnp.int32)
    srcp = jnp.concatenate([edge_index[0].astype(jnp.int32), loop, padi])
    dstp = jnp.concatenate([edge_index[1].astype(jnp.int32), loop, padi])

    Am1, Bm1 = _att_mats(a_src1, a_dst1, HEADS, HID)
    Am2, Bm2 = _att_mats(a_src2, a_dst2, 1, HID)

    p0, p1, p2, p3, A1r, B1r = _tc1(x, W1, Am1, Bm1)
    padA = jnp.full((NP - N, 16), NEG, jnp.float32)
    padH = jnp.zeros((NP - N, 128), jnp.float32)
    A1 = jnp.concatenate([A1r, padA])
    B1 = jnp.concatenate([B1r, padA])
    hflat = jnp.concatenate([p0, padH, p1, padH, p2, padH, p3, padH])

    ex1, den1 = _att(srcp, dstp, A1, B1)
    al1 = _alpha(dstp, ex1, den1[0], den1[1])
    out1 = _msg1(srcp, dstp, hflat, al1)

    h2r, A2r, B2r = _tc2(out1[0, :N], out1[1, :N], out1[2, :N], out1[3, :N],
                         b1, W2, Am2, Bm2)
    h2 = jnp.concatenate([h2r, jnp.zeros((NP - N, HID), jnp.float32)])
    A2 = jnp.concatenate([A2r, padA])
    B2 = jnp.concatenate([B2r, padA])

    ex2, den2 = _att(srcp, dstp, A2, B2)
    al2 = _alpha(dstp, ex2, den2[0], den2[1])
    out2 = _msg2(srcp, dstp, h2, al2)

    emb, logits, soft, hard = _tc3(out2[0, :N], out2[1, :N], b2, Wc, bc)
    return (logits, emb, soft, hard[:, 0])
